# baseline (speedup 1.0000x reference)
import functools

import jax
import jax.numpy as jnp
from jax import lax
from jax.experimental import pallas as pl
from jax.experimental.pallas import tpu as pltpu
from jax.experimental.pallas import tpu_sc as plsc

NUM_CLASSES = 100000
TEXT_DIM = 2048
HID = 256
CNT = 5
B = 4096
TOTAL = B * CNT

_SC_INFO = plsc.get_sparse_core_info()
_NC = _SC_INFO.num_cores
_NS = _SC_INFO.num_subcores
_NW = _NC * _NS
_K = 16

_NSLICE = 4
_S = TOTAL // _NSLICE


def _make_sc_gather(rows):
    bpw = rows // _NW
    nchunk = bpw // _K
    nhalf = nchunk // 2

    @functools.partial(
        pl.kernel,
        mesh=plsc.VectorSubcoreMesh(core_axis_name="c", subcore_axis_name="s"),
        out_type=jax.ShapeDtypeStruct((rows, TEXT_DIM), jnp.float32),
        scratch_types=[
            pltpu.VMEM((bpw,), jnp.int32),
            pltpu.VMEM((_K, TEXT_DIM), jnp.float32),
            pltpu.VMEM((_K, TEXT_DIM), jnp.float32),
            pltpu.SemaphoreType.DMA,
            pltpu.SemaphoreType.DMA,
            pltpu.SemaphoreType.DMA,
            pltpu.SemaphoreType.DMA,
        ],
    )
    def _sc_gather(table_hbm, idx_hbm, out_hbm, idx_v, buf0, buf1,
                   gsem0, gsem1, wsem0, wsem1):
        wid = lax.axis_index("s") * _NC + lax.axis_index("c")
        base = wid * bpw
        pltpu.sync_copy(idx_hbm.at[pl.ds(base, bpw)], idx_v)

        def _gather(c, buf, sem):
            pltpu.async_copy(table_hbm.at[idx_v.at[pl.ds(c * _K, _K)]], buf, sem)

        def _wait(buf, sem):
            pltpu.make_async_copy(buf, out_hbm.at[pl.ds(base, _K)], sem).wait()

        _gather(0, buf0, gsem0)

        def body(i, carry):
            c0 = 2 * i

            @pl.when(i > 0)
            def _():
                _wait(buf1, wsem1)

            _gather(c0 + 1, buf1, gsem1)
            pltpu.make_async_copy(
                table_hbm.at[idx_v.at[pl.ds(0, _K)]], buf0, gsem0
            ).wait()
            pltpu.async_copy(buf0, out_hbm.at[pl.ds(base + c0 * _K, _K)], wsem0)

            @pl.when(i < nhalf - 1)
            def _():
                _wait(buf0, wsem0)
                _gather(c0 + 2, buf0, gsem0)

            pltpu.make_async_copy(
                table_hbm.at[idx_v.at[pl.ds(0, _K)]], buf1, gsem1
            ).wait()
            pltpu.async_copy(
                buf1, out_hbm.at[pl.ds(base + (c0 + 1) * _K, _K)], wsem1
            )
            return carry

        lax.fori_loop(0, nhalf, body, 0)
        _wait(buf0, wsem0)
        _wait(buf1, wsem1)

    return _sc_gather


_BM = 1024


def _mlp_body(e_ref, w1_ref, b1_ref, w2_ref, b2_ref, o_ref):
    h = jnp.dot(e_ref[...], w1_ref[...], preferred_element_type=jnp.float32)
    h = h + b1_ref[...]
    h = h * lax.logistic(h)
    o = jnp.dot(h, w2_ref[...], preferred_element_type=jnp.float32)
    o_ref[...] = o + b2_ref[...]


def _mlp(e, w1, b1, w2, b2):
    rows = e.shape[0]
    return pl.pallas_call(
        _mlp_body,
        grid=(rows // _BM,),
        in_specs=[
            pl.BlockSpec((_BM, TEXT_DIM), lambda i: (i, 0)),
            pl.BlockSpec((TEXT_DIM, HID), lambda i: (0, 0)),
            pl.BlockSpec((1, HID), lambda i: (0, 0)),
            pl.BlockSpec((HID, HID), lambda i: (0, 0)),
            pl.BlockSpec((1, HID), lambda i: (0, 0)),
        ],
        out_specs=pl.BlockSpec((_BM, HID), lambda i: (i, 0)),
        out_shape=jax.ShapeDtypeStruct((rows, HID), jnp.float32),
    )(e, w1, b1, w2, b2)


_sc_gather_slice = _make_sc_gather(_S)


def kernel(label_ids, prompt_embeds, W1, b1, W2, b2):
    ids = label_ids.reshape(-1).astype(jnp.int32)
    b1r = b1.reshape(1, HID)
    b2r = b2.reshape(1, HID)
    outs = []
    for j in range(_NSLICE):
        g = _sc_gather_slice(prompt_embeds, lax.slice(ids, (j * _S,), ((j + 1) * _S,)))
        outs.append(_mlp(g, W1, b1r, W2, b2r))
    out = jnp.concatenate(outs, axis=0)
    return out.reshape(B, CNT * HID)


# speedup vs baseline: 1.2231x; 1.2231x over previous
import functools

import jax
import jax.numpy as jnp
from jax import lax
from jax.experimental import pallas as pl
from jax.experimental.pallas import tpu as pltpu
from jax.experimental.pallas import tpu_sc as plsc

NUM_CLASSES = 100000
TEXT_DIM = 2048
HID = 256
CNT = 5
B = 4096
TOTAL = B * CNT
OUT_D = CNT * HID

_SC_INFO = plsc.get_sparse_core_info()
_NC = _SC_INFO.num_cores
_NS = _SC_INFO.num_subcores
_NW = _NC * _NS
_K = 16

_NSLICE = 4
_BEX = B // _NSLICE
_S = _BEX * CNT
_BM_EX = 256


def _make_sc_gather(rows):
    bpw = rows // _NW
    nchunk = bpw // _K
    nhalf = nchunk // 2

    @functools.partial(
        pl.kernel,
        mesh=plsc.VectorSubcoreMesh(core_axis_name="c", subcore_axis_name="s"),
        out_type=jax.ShapeDtypeStruct((rows, TEXT_DIM), jnp.float32),
        scratch_types=[
            pltpu.VMEM((bpw,), jnp.int32),
            pltpu.VMEM((_K, TEXT_DIM), jnp.float32),
            pltpu.VMEM((_K, TEXT_DIM), jnp.float32),
            pltpu.SemaphoreType.DMA,
            pltpu.SemaphoreType.DMA,
            pltpu.SemaphoreType.DMA,
            pltpu.SemaphoreType.DMA,
        ],
    )
    def _sc_gather(table_hbm, idx_hbm, out_hbm, idx_v, buf0, buf1,
                   gsem0, gsem1, wsem0, wsem1):
        wid = lax.axis_index("s") * _NC + lax.axis_index("c")
        base = wid * bpw
        pltpu.sync_copy(idx_hbm.at[pl.ds(base, bpw)], idx_v)

        def _gather(c, buf, sem):
            pltpu.async_copy(table_hbm.at[idx_v.at[pl.ds(c * _K, _K)]], buf, sem)

        def _wait(buf, sem):
            pltpu.make_async_copy(buf, out_hbm.at[pl.ds(base, _K)], sem).wait()

        _gather(0, buf0, gsem0)

        def body(i, carry):
            c0 = 2 * i

            @pl.when(i > 0)
            def _():
                _wait(buf1, wsem1)

            _gather(c0 + 1, buf1, gsem1)
            pltpu.make_async_copy(
                table_hbm.at[idx_v.at[pl.ds(0, _K)]], buf0, gsem0
            ).wait()
            pltpu.async_copy(buf0, out_hbm.at[pl.ds(base + c0 * _K, _K)], wsem0)

            @pl.when(i < nhalf - 1)
            def _():
                _wait(buf0, wsem0)
                _gather(c0 + 2, buf0, gsem0)

            pltpu.make_async_copy(
                table_hbm.at[idx_v.at[pl.ds(0, _K)]], buf1, gsem1
            ).wait()
            pltpu.async_copy(
                buf1, out_hbm.at[pl.ds(base + (c0 + 1) * _K, _K)], wsem1
            )
            return carry

        lax.fori_loop(0, nhalf, body, 0)
        _wait(buf0, wsem0)
        _wait(buf1, wsem1)

    return _sc_gather


_sc_gather_slice = _make_sc_gather(_S)


def _mlp_body(*refs):
    e_refs = refs[:CNT]
    if len(refs) == CNT + 6:
        w1_ref, b1_ref, w2_ref, b2_ref, _acc_ref, o_ref = refs[CNT:]
    else:
        w1_ref, b1_ref, w2_ref, b2_ref, o_ref = refs[CNT:]
    w1 = w1_ref[...]
    w2 = w2_ref[...]
    b1 = b1_ref[...]
    b2 = b2_ref[...]
    for t in range(CNT):
        et = e_refs[t][0]
        h = jnp.dot(et, w1, preferred_element_type=jnp.float32) + b1
        h = h * lax.logistic(h)
        o_ref[:, t * HID:(t + 1) * HID] = (
            jnp.dot(h, w2, preferred_element_type=jnp.float32) + b2
        )


def _mlp_slice(j, acc, e3, w1, b1, w2, b2):
    nblk = _BEX // _BM_EX
    base_blk = j * nblk

    e_specs = [
        pl.BlockSpec((1, _BM_EX, TEXT_DIM),
                     functools.partial(lambda t, i: (t, i, 0), t))
        for t in range(CNT)
    ]
    w_specs = [
        pl.BlockSpec((TEXT_DIM, HID), lambda i: (0, 0)),
        pl.BlockSpec((1, HID), lambda i: (0, 0)),
        pl.BlockSpec((HID, HID), lambda i: (0, 0)),
        pl.BlockSpec((1, HID), lambda i: (0, 0)),
    ]
    if acc is None:
        return pl.pallas_call(
            _mlp_body,
            grid=(nblk,),
            in_specs=e_specs + w_specs,
            out_specs=pl.BlockSpec((_BM_EX, OUT_D), lambda i: (base_blk + i, 0)),
            out_shape=jax.ShapeDtypeStruct((B, OUT_D), jnp.float32),
        )(*([e3] * CNT), w1, b1, w2, b2)
    return pl.pallas_call(
        _mlp_body,
        grid=(nblk,),
        in_specs=e_specs + w_specs + [pl.BlockSpec(memory_space=pl.ANY)],
        out_specs=pl.BlockSpec((_BM_EX, OUT_D), lambda i: (base_blk + i, 0)),
        out_shape=jax.ShapeDtypeStruct((B, OUT_D), jnp.float32),
        input_output_aliases={CNT + 4: 0},
    )(*([e3] * CNT), w1, b1, w2, b2, acc)


def kernel(label_ids, prompt_embeds, W1, b1, W2, b2):
    ids = label_ids.astype(jnp.int32)
    b1r = b1.reshape(1, HID)
    b2r = b2.reshape(1, HID)
    acc = None
    for j in range(_NSLICE):
        ids_j = ids[j * _BEX:(j + 1) * _BEX, :].T.reshape(-1)
        g = _sc_gather_slice(prompt_embeds, ids_j)
        g3 = g.reshape(CNT, _BEX, TEXT_DIM)
        acc = _mlp_slice(j, acc, g3, W1, b1r, W2, b2r)
    return acc
